# baseline (speedup 1.0000x reference)
import jax
import jax.numpy as jnp
from jax import lax
from jax.experimental import pallas as pl
from jax.experimental.pallas import tpu as pltpu
from jax.experimental.pallas import tpu_sc as plsc

NV = 50000
NN = 50176
PT = 3136
W = 224
NK = PT // W
W2 = 112
NK2 = PT // W2
E = 800000
ECH = 128
F = 4
NB = 100
CT = NB * F
EPAD = 16 * (CT + F) * ECH
TRASH = NV


def _body(ei, embp, z2, z1, out, g0, g1, g2,
          acc_sh, deg_sh, rvN, dvv, rowv, colv, onesv,
          isem, gsem, ssem, zsem, lsem):
    c = lax.axis_index("c")
    s = lax.axis_index("s")
    coffv = jnp.full((16,), c * NN, jnp.int32)

    def idx_src(which, ct):
        return ei.at[which, pl.ds((s + 16 * ct) * ECH, ECH)]

    def slotref(f):
        return rvN.at[pl.ds(f * ECH, ECH), :]

    def prefetch(g, h):
        ds = []
        for f in range(F):
            ct = g * F + f
            ds.append(pltpu.async_copy(idx_src(0, ct), rowv.at[h + f], isem))
            ds.append(pltpu.async_copy(idx_src(1, ct), colv.at[h + f], isem))
        return ds

    for t in range(8):
        onesv[0, pl.ds(16 * t, 16)] = jnp.full((16,), 1.0, jnp.float32)
    pltpu.sync_copy(z1, deg_sh.at[pl.ds(s * PT, PT)])
    plsc.subcore_barrier()

    for f in range(F):
        pltpu.sync_copy(idx_src(0, f), rowv.at[f])
        pltpu.sync_copy(idx_src(1, f), colv.at[f])

    def deg_step(g, _):
        h = lax.rem(g, 2) * F
        hn = F - h
        di = prefetch(g + 1, hn)
        sd = []
        for f in range(F):
            sd.append(pltpu.async_copy(
                onesv.at[0], deg_sh.at[rowv.at[h + f]], ssem, add=True))
            sd.append(pltpu.async_copy(
                onesv.at[0], deg_sh.at[colv.at[h + f]], ssem, add=True))
        for d in sd:
            d.wait()
        for d in di:
            d.wait()
        return 0

    lax.fori_loop(0, NB, deg_step, 0)
    plsc.subcore_barrier()

    def init_step(k, _):
        base = s * PT + k * W
        gb = c * NN + base
        ev = rvN.at[pl.ds(0, W), :]
        zd = pltpu.async_copy(z2, acc_sh.at[pl.ds(base, W), :], zsem)
        pltpu.sync_copy(deg_sh.at[pl.ds(base, W)], dvv)

        def rsqrt_step(gg, _):
            x = jnp.maximum(dvv[pl.ds(gg * 16, 16)], 1.0)
            bits = lax.bitcast_convert_type(x, jnp.int32)
            y = lax.bitcast_convert_type(
                jnp.int32(0x5F3759DF) - lax.shift_right_arithmetic(bits, 1),
                jnp.float32)
            half = x * 0.5
            for _ in range(3):
                y = y * (1.5 - half * y * y)
            dvv[pl.ds(gg * 16, 16)] = y
            return 0

        lax.fori_loop(0, W // 16, rsqrt_step, 0)
        pltpu.sync_copy(dvv, deg_sh.at[pl.ds(base, W)])
        pltpu.sync_copy(embp.at[pl.ds(gb, W), :], ev)

        def scale_step(gg, _):
            d16 = dvv[pl.ds(gg * 16, 16)]
            for i in range(16):
                n = gg * 16 + i
                d = jnp.full((16,), d16[i])
                ev[n, pl.ds(0, 16)] = ev[n, pl.ds(0, 16)] * d
                ev[n, pl.ds(16, 16)] = ev[n, pl.ds(16, 16)] * d
            return 0

        lax.fori_loop(0, W // 16, scale_step, 0)
        pltpu.sync_copy(ev, g0.at[pl.ds(gb, W), :])
        zd.wait()
        return 0

    lax.fori_loop(0, NK, init_step, 0)
    plsc.subcore_barrier()

    for layer, (gin, gout) in enumerate([(g0, g1), (g1, g2), (g2, None)]):
        last = gout is None

        for f in range(F):
            pltpu.sync_copy(idx_src(0, f), rowv.at[f])
            pltpu.sync_copy(idx_src(1, f), colv.at[f])

        def edge_step(g, _):
            h = lax.rem(g, 2) * F
            hn = F - h
            di = prefetch(g + 1, hn)
            gd = []
            for f in range(F):
                for t in range(8):
                    colv[h + f, pl.ds(16 * t, 16)] = (
                        colv[h + f, pl.ds(16 * t, 16)] + coffv)
                gd.append(pltpu.async_copy(
                    gin.at[colv.at[h + f]], slotref(f), gsem))
            sd = []
            for f in range(F):
                gd[f].wait()
                sd.append(pltpu.async_copy(
                    slotref(f), acc_sh.at[rowv.at[h + f]], ssem, add=True))
            for d in sd:
                d.wait()
            for d in di:
                d.wait()
            return 0

        lax.fori_loop(0, NB, edge_step, 0)
        plsc.subcore_barrier()

        if not last:
            def write_step(k, _):
                base = s * PT + k * W
                gb = c * NN + base
                sv = rvN.at[pl.ds(0, W), :]
                pltpu.sync_copy(acc_sh.at[pl.ds(base, W), :], sv)
                zd = pltpu.async_copy(z2, acc_sh.at[pl.ds(base, W), :], zsem)
                pltpu.sync_copy(deg_sh.at[pl.ds(base, W)], dvv)

                def out_step(gg, _):
                    d16 = dvv[pl.ds(gg * 16, 16)]
                    d16 = d16 * d16
                    for i in range(16):
                        n = gg * 16 + i
                        d = jnp.full((16,), d16[i])
                        sv[n, pl.ds(0, 16)] = sv[n, pl.ds(0, 16)] * d
                        sv[n, pl.ds(16, 16)] = sv[n, pl.ds(16, 16)] * d
                    return 0

                lax.fori_loop(0, W // 16, out_step, 0)
                pltpu.sync_copy(sv, gout.at[pl.ds(gb, W), :])
                zd.wait()
                return 0

            lax.fori_loop(0, NK, write_step, 0)
            plsc.subcore_barrier()
        else:
            def final_step(k, _):
                base = s * PT + k * W2
                gb = c * NN + base
                sv = rvN.at[pl.ds(0, W2), :]
                ev = rvN.at[pl.ds(W2, W2), :]
                bv1 = rvN.at[pl.ds(2 * W2, W2), :]
                bv2 = rvN.at[pl.ds(3 * W2, W2), :]
                pltpu.sync_copy(acc_sh.at[pl.ds(base, W2), :], sv)
                pltpu.sync_copy(embp.at[pl.ds(gb, W2), :], ev)
                pltpu.sync_copy(g1.at[pl.ds(gb, W2), :], bv1)
                pltpu.sync_copy(g2.at[pl.ds(gb, W2), :], bv2)
                pltpu.sync_copy(deg_sh.at[pl.ds(base, W2)],
                                dvv.at[pl.ds(0, W2)])

                def out_step(gg, _):
                    d16 = dvv[pl.ds(gg * 16, 16)]
                    q16 = 0.25 / d16
                    d16q = d16 * 0.25
                    for i in range(16):
                        n = gg * 16 + i
                        d = jnp.full((16,), d16q[i])
                        q = jnp.full((16,), q16[i])
                        for h in (0, 16):
                            ev[n, pl.ds(h, 16)] = (
                                ev[n, pl.ds(h, 16)] * 0.25
                                + (bv1[n, pl.ds(h, 16)]
                                   + bv2[n, pl.ds(h, 16)]) * q
                                + sv[n, pl.ds(h, 16)] * d)
                    return 0

                lax.fori_loop(0, W2 // 16, out_step, 0)
                pltpu.sync_copy(ev, out.at[pl.ds(gb, W2), :])
                return 0

            lax.fori_loop(0, NK2, final_step, 0)


_mesh = plsc.VectorSubcoreMesh(core_axis_name="c", subcore_axis_name="s")

_sc_call = pl.kernel(
    _body,
    out_type=(
        jax.ShapeDtypeStruct((2 * NN, 32), jnp.float32),
        jax.ShapeDtypeStruct((2 * NN, 32), jnp.float32),
        jax.ShapeDtypeStruct((2 * NN, 32), jnp.float32),
        jax.ShapeDtypeStruct((2 * NN, 32), jnp.float32),
    ),
    mesh=_mesh,
    compiler_params=pltpu.CompilerParams(use_tc_tiling_on_sc=False),
    scratch_types=[
        pltpu.VMEM_SHARED((NN, 32), jnp.float32),
        pltpu.VMEM_SHARED((NN,), jnp.float32),
        pltpu.VMEM((F * ECH, 32), jnp.float32),
        pltpu.VMEM((W,), jnp.float32),
        pltpu.VMEM((2 * F, ECH), jnp.int32),
        pltpu.VMEM((2 * F, ECH), jnp.int32),
        pltpu.VMEM((1, ECH), jnp.float32),
        pltpu.SemaphoreType.DMA,
        pltpu.SemaphoreType.DMA,
        pltpu.SemaphoreType.DMA,
        pltpu.SemaphoreType.DMA,
        pltpu.SemaphoreType.DMA,
    ],
)


@jax.jit
def kernel(edge_index, embedding_weight):
    eip = jnp.pad(edge_index, ((0, 0), (0, EPAD - E)),
                  constant_values=TRASH)
    embp = (jnp.zeros((2 * NN, 32), jnp.float32)
            .at[:NV].set(embedding_weight[:, :32])
            .at[NN:NN + NV].set(embedding_weight[:, 32:]))
    z2 = jnp.zeros((W, 32), jnp.float32)
    z1 = jnp.zeros((PT,), jnp.float32)
    outf, _, _, _ = _sc_call(eip, embp, z2, z1)
    final = jnp.concatenate([outf[:NV], outf[NN:NN + NV]], axis=1)
    return final[:NV // 2], final[NV // 2:]


# speedup vs baseline: 1.5280x; 1.5280x over previous
import jax
import jax.numpy as jnp
from jax import lax
from jax.experimental import pallas as pl
from jax.experimental.pallas import tpu as pltpu
from jax.experimental.pallas import tpu_sc as plsc

NV = 50000
NN = 50176
PT = 3136
W = 224
NK = PT // W
W2 = 112
NK2 = PT // W2
E = 800000
ECH = 128
F = 4
NB = 100
CT = NB * F
EPAD = 16 * (CT + F) * ECH
TRASH = NV


def _body(ei, embp, z2, z1, out, g0, g1, g2,
          acc_sh, deg_sh, rvN, dvv, rowv, colv, onesv,
          isem, gsem, ssem, zsem, lsem):
    c = lax.axis_index("c")
    s = lax.axis_index("s")
    coffv = jnp.full((16,), c * NN, jnp.int32)

    def idx_src(which, ct):
        return ei.at[which, pl.ds((s + 16 * ct) * ECH, ECH)]

    def slotref(f):
        return rvN.at[pl.ds(f * ECH, ECH), :]

    def prefetch(g, h):
        ds = []
        for f in range(F):
            ct = g * F + f
            ds.append(pltpu.async_copy(idx_src(0, ct), rowv.at[h + f], isem))
            ds.append(pltpu.async_copy(idx_src(1, ct), colv.at[h + f], isem))
        return ds

    for t in range(8):
        onesv[0, pl.ds(16 * t, 16)] = jnp.full((16,), 1.0, jnp.float32)
    pltpu.sync_copy(z1, deg_sh.at[pl.ds(s * PT, PT)])
    plsc.subcore_barrier()

    for f in range(F):
        pltpu.sync_copy(idx_src(0, f), rowv.at[f])
        pltpu.sync_copy(idx_src(1, f), colv.at[f])

    def deg_step(g, _):
        h = lax.rem(g, 2) * F
        hn = F - h
        di = prefetch(g + 1, hn)
        sd = []
        for f in range(F):
            sd.append(pltpu.async_copy(
                onesv.at[0], deg_sh.at[rowv.at[h + f]], ssem, add=True))
            sd.append(pltpu.async_copy(
                onesv.at[0], deg_sh.at[colv.at[h + f]], ssem, add=True))
        for d in sd:
            d.wait()
        for d in di:
            d.wait()
        return 0

    lax.fori_loop(0, NB, deg_step, 0)
    plsc.subcore_barrier()

    def init_step(k, _):
        base = s * PT + k * W
        gb = c * NN + base
        ev = rvN.at[pl.ds(0, W), :]
        zd = pltpu.async_copy(z2, acc_sh.at[pl.ds(base, W), :], zsem)
        pltpu.sync_copy(deg_sh.at[pl.ds(base, W)], dvv)

        def rsqrt_step(gg, _):
            x = jnp.maximum(dvv[pl.ds(gg * 16, 16)], 1.0)
            bits = lax.bitcast_convert_type(x, jnp.int32)
            y = lax.bitcast_convert_type(
                jnp.int32(0x5F3759DF) - lax.shift_right_arithmetic(bits, 1),
                jnp.float32)
            half = x * 0.5
            for _ in range(3):
                y = y * (1.5 - half * y * y)
            dvv[pl.ds(gg * 16, 16)] = y
            return 0

        lax.fori_loop(0, W // 16, rsqrt_step, 0)
        pltpu.sync_copy(dvv, deg_sh.at[pl.ds(base, W)])
        pltpu.sync_copy(embp.at[pl.ds(gb, W), :], ev)

        def scale_step(gg, _):
            d16 = dvv[pl.ds(gg * 16, 16)]
            for i in range(16):
                n = gg * 16 + i
                d = jnp.full((16,), d16[i])
                ev[n, pl.ds(0, 16)] = ev[n, pl.ds(0, 16)] * d
                ev[n, pl.ds(16, 16)] = ev[n, pl.ds(16, 16)] * d
            return 0

        lax.fori_loop(0, W // 16, scale_step, 0)
        pltpu.sync_copy(ev, g0.at[pl.ds(gb, W), :])
        zd.wait()
        return 0

    lax.fori_loop(0, NK, init_step, 0)
    plsc.subcore_barrier()

    for layer, (gin, gout) in enumerate([(g0, g1), (g1, g2), (g2, None)]):
        last = gout is None

        for f in range(F):
            pltpu.sync_copy(idx_src(0, f), rowv.at[f])
            pltpu.sync_copy(idx_src(1, f), colv.at[f])

        def edge_step(g, _):
            h = lax.rem(g, 2) * F
            hn = F - h
            di = prefetch(g + 1, hn)
            gd = []
            for f in range(F):
                for t in range(8):
                    colv[h + f, pl.ds(16 * t, 16)] = (
                        colv[h + f, pl.ds(16 * t, 16)] + coffv)
                gd.append(pltpu.async_copy(
                    gin.at[colv.at[h + f]], slotref(f), gsem))
            sd = []
            for f in range(F):
                gd[f].wait()
                sd.append(pltpu.async_copy(
                    slotref(f), acc_sh.at[rowv.at[h + f]], ssem, add=True))
            for d in sd:
                d.wait()
            for d in di:
                d.wait()
            return 0

        lax.fori_loop(0, NB, edge_step, 0)
        plsc.subcore_barrier()

        if not last:
            def write_step(k, _):
                base = s * PT + k * W
                gb = c * NN + base
                sv = rvN.at[pl.ds(0, W), :]
                pltpu.sync_copy(acc_sh.at[pl.ds(base, W), :], sv)
                zd = pltpu.async_copy(z2, acc_sh.at[pl.ds(base, W), :], zsem)
                pltpu.sync_copy(deg_sh.at[pl.ds(base, W)], dvv)

                def out_step(gg, _):
                    d16 = dvv[pl.ds(gg * 16, 16)]
                    d16 = d16 * d16
                    for i in range(16):
                        n = gg * 16 + i
                        d = jnp.full((16,), d16[i])
                        sv[n, pl.ds(0, 16)] = sv[n, pl.ds(0, 16)] * d
                        sv[n, pl.ds(16, 16)] = sv[n, pl.ds(16, 16)] * d
                    return 0

                lax.fori_loop(0, W // 16, out_step, 0)
                pltpu.sync_copy(sv, gout.at[pl.ds(gb, W), :])
                zd.wait()
                return 0

            lax.fori_loop(0, NK, write_step, 0)
            plsc.subcore_barrier()
        else:
            def final_step(k, _):
                base = s * PT + k * W2
                gb = c * NN + base
                sv = rvN.at[pl.ds(0, W2), :]
                ev = rvN.at[pl.ds(W2, W2), :]
                bv1 = rvN.at[pl.ds(2 * W2, W2), :]
                bv2 = rvN.at[pl.ds(3 * W2, W2), :]
                pltpu.sync_copy(acc_sh.at[pl.ds(base, W2), :], sv)
                pltpu.sync_copy(embp.at[pl.ds(gb, W2), :], ev)
                pltpu.sync_copy(g1.at[pl.ds(gb, W2), :], bv1)
                pltpu.sync_copy(g2.at[pl.ds(gb, W2), :], bv2)
                pltpu.sync_copy(deg_sh.at[pl.ds(base, W2)],
                                dvv.at[pl.ds(0, W2)])

                def out_step(gg, _):
                    d16 = dvv[pl.ds(gg * 16, 16)]
                    q16 = 0.25 / d16
                    d16q = d16 * 0.25
                    for i in range(16):
                        n = gg * 16 + i
                        d = jnp.full((16,), d16q[i])
                        q = jnp.full((16,), q16[i])
                        for h in (0, 16):
                            ev[n, pl.ds(h, 16)] = (
                                ev[n, pl.ds(h, 16)] * 0.25
                                + (bv1[n, pl.ds(h, 16)]
                                   + bv2[n, pl.ds(h, 16)]) * q
                                + sv[n, pl.ds(h, 16)] * d)
                    return 0

                lax.fori_loop(0, W2 // 16, out_step, 0)
                pltpu.sync_copy(ev, out.at[pl.ds(gb, W2), :])
                return 0

            lax.fori_loop(0, NK2, final_step, 0)


_mesh = plsc.VectorSubcoreMesh(core_axis_name="c", subcore_axis_name="s")

_sc_call = pl.kernel(
    _body,
    out_type=(
        jax.ShapeDtypeStruct((2 * NN, 32), jnp.float32),
        jax.ShapeDtypeStruct((2 * NN, 32), jnp.float32),
        jax.ShapeDtypeStruct((2 * NN, 32), jnp.float32),
        jax.ShapeDtypeStruct((2 * NN, 32), jnp.float32),
    ),
    mesh=_mesh,
    compiler_params=pltpu.CompilerParams(use_tc_tiling_on_sc=False),
    scratch_types=[
        pltpu.VMEM_SHARED((NN, 32), jnp.float32),
        pltpu.VMEM_SHARED((NN,), jnp.float32),
        pltpu.VMEM((F * ECH, 32), jnp.float32),
        pltpu.VMEM((W,), jnp.float32),
        pltpu.VMEM((2 * F, ECH), jnp.int32),
        pltpu.VMEM((2 * F, ECH), jnp.int32),
        pltpu.VMEM((1, ECH), jnp.float32),
        pltpu.SemaphoreType.DMA,
        pltpu.SemaphoreType.DMA,
        pltpu.SemaphoreType.DMA,
        pltpu.SemaphoreType.DMA,
        pltpu.SemaphoreType.DMA,
    ],
)


@jax.jit
def kernel(edge_index, embedding_weight):
    padv = TRASH + (jnp.arange(EPAD - E, dtype=jnp.int32) % 176)
    eip = jnp.concatenate(
        [edge_index, jnp.stack([padv, padv])], axis=1)
    embp = (jnp.zeros((2 * NN, 32), jnp.float32)
            .at[:NV].set(embedding_weight[:, :32])
            .at[NN:NN + NV].set(embedding_weight[:, 32:]))
    z2 = jnp.zeros((W, 32), jnp.float32)
    z1 = jnp.zeros((PT,), jnp.float32)
    outf, _, _, _ = _sc_call(eip, embp, z2, z1)
    final = jnp.concatenate([outf[:NV], outf[NN:NN + NV]], axis=1)
    return final[:NV // 2], final[NV // 2:]


# speedup vs baseline: 1.7833x; 1.1671x over previous
import jax
import jax.numpy as jnp
from jax import lax
from jax.experimental import pallas as pl
from jax.experimental.pallas import tpu as pltpu
from jax.experimental.pallas import tpu_sc as plsc

NV = 50000
NN = 50176
PT = 3136
W = 224
NK = PT // W
W2 = 112
NK2 = PT // W2
E = 800000
ECH = 128
F = 4
NB = 98
CT = NB * F
EPAD = 16 * (CT + F) * ECH
TRASH = NV


def _body(ei, embp, z2, z1, out, g0, g1, g2,
          acc_sh, deg_sh, rvN, dvv, rowv, colv, onesv,
          isem, gsem, ssem, zsem, lsem):
    c = lax.axis_index("c")
    s = lax.axis_index("s")
    coffv = jnp.full((16,), c * NN, jnp.int32)

    def idx_src(which, ct):
        return ei.at[which, pl.ds((s + 16 * ct) * ECH, ECH)]

    def slotref(f):
        return rvN.at[pl.ds(f * ECH, ECH), :]

    def prefetch(g, h):
        ds = []
        for f in range(F):
            ct = g * F + f
            ds.append(pltpu.async_copy(idx_src(0, ct), rowv.at[h + f], isem))
            ds.append(pltpu.async_copy(idx_src(1, ct), colv.at[h + f], isem))
        return ds

    for t in range(8):
        onesv[0, pl.ds(16 * t, 16)] = jnp.full((16,), 1.0, jnp.float32)
    pltpu.sync_copy(z1, deg_sh.at[pl.ds(s * PT, PT)])
    plsc.subcore_barrier()

    for f in range(F):
        pltpu.sync_copy(idx_src(0, f), rowv.at[f])
        pltpu.sync_copy(idx_src(1, f), colv.at[f])

    def deg_step(g, _):
        h = lax.rem(g, 2) * F
        hn = F - h
        di = prefetch(g + 1, hn)
        sd = []
        for f in range(F):
            sd.append(pltpu.async_copy(
                onesv.at[0], deg_sh.at[rowv.at[h + f]], ssem, add=True))
            sd.append(pltpu.async_copy(
                onesv.at[0], deg_sh.at[colv.at[h + f]], ssem, add=True))
        for d in sd:
            d.wait()
        for d in di:
            d.wait()
        return 0

    lax.fori_loop(0, NB, deg_step, 0)
    plsc.subcore_barrier()

    def init_step(k, _):
        base = s * PT + k * W
        gb = c * NN + base
        ev = rvN.at[pl.ds(0, W), :]
        zd = pltpu.async_copy(z2, acc_sh.at[pl.ds(base, W), :], zsem)
        pltpu.sync_copy(deg_sh.at[pl.ds(base, W)], dvv)

        def rsqrt_step(gg, _):
            x = jnp.maximum(dvv[pl.ds(gg * 16, 16)], 1.0)
            bits = lax.bitcast_convert_type(x, jnp.int32)
            y = lax.bitcast_convert_type(
                jnp.int32(0x5F3759DF) - lax.shift_right_arithmetic(bits, 1),
                jnp.float32)
            half = x * 0.5
            for _ in range(3):
                y = y * (1.5 - half * y * y)
            dvv[pl.ds(gg * 16, 16)] = y
            return 0

        lax.fori_loop(0, W // 16, rsqrt_step, 0)
        pltpu.sync_copy(dvv, deg_sh.at[pl.ds(base, W)])
        pltpu.sync_copy(embp.at[pl.ds(base, W), pl.ds(c * 32, 32)], ev)

        def scale_step(gg, _):
            d16 = dvv[pl.ds(gg * 16, 16)]
            for i in range(16):
                n = gg * 16 + i
                d = jnp.full((16,), d16[i])
                ev[n, pl.ds(0, 16)] = ev[n, pl.ds(0, 16)] * d
                ev[n, pl.ds(16, 16)] = ev[n, pl.ds(16, 16)] * d
            return 0

        lax.fori_loop(0, W // 16, scale_step, 0)
        pltpu.sync_copy(ev, g0.at[pl.ds(gb, W), :])
        zd.wait()
        return 0

    lax.fori_loop(0, NK, init_step, 0)
    plsc.subcore_barrier()

    for layer, (gin, gout) in enumerate([(g0, g1), (g1, g2), (g2, None)]):
        last = gout is None

        for f in range(F):
            pltpu.sync_copy(idx_src(0, f), rowv.at[f])
            pltpu.sync_copy(idx_src(1, f), colv.at[f])

        def edge_step(g, _):
            h = lax.rem(g, 2) * F
            hn = F - h
            di = prefetch(g + 1, hn)
            gd = []
            for f in range(F):
                for t in range(8):
                    colv[h + f, pl.ds(16 * t, 16)] = (
                        colv[h + f, pl.ds(16 * t, 16)] + coffv)
                gd.append(pltpu.async_copy(
                    gin.at[colv.at[h + f]], slotref(f), gsem))
            sd = []
            for f in range(F):
                gd[f].wait()
                sd.append(pltpu.async_copy(
                    slotref(f), acc_sh.at[rowv.at[h + f]], ssem, add=True))
            for d in sd:
                d.wait()
            for d in di:
                d.wait()
            return 0

        lax.fori_loop(0, NB, edge_step, 0)
        plsc.subcore_barrier()

        if not last:
            def write_step(k, _):
                base = s * PT + k * W
                gb = c * NN + base
                sv = rvN.at[pl.ds(0, W), :]
                pltpu.sync_copy(acc_sh.at[pl.ds(base, W), :], sv)
                zd = pltpu.async_copy(z2, acc_sh.at[pl.ds(base, W), :], zsem)
                pltpu.sync_copy(deg_sh.at[pl.ds(base, W)], dvv)

                def out_step(gg, _):
                    d16 = dvv[pl.ds(gg * 16, 16)]
                    d16 = d16 * d16
                    for i in range(16):
                        n = gg * 16 + i
                        d = jnp.full((16,), d16[i])
                        sv[n, pl.ds(0, 16)] = sv[n, pl.ds(0, 16)] * d
                        sv[n, pl.ds(16, 16)] = sv[n, pl.ds(16, 16)] * d
                    return 0

                lax.fori_loop(0, W // 16, out_step, 0)
                pltpu.sync_copy(sv, gout.at[pl.ds(gb, W), :])
                zd.wait()
                return 0

            lax.fori_loop(0, NK, write_step, 0)
            plsc.subcore_barrier()
        else:
            def final_step(k, _):
                base = s * PT + k * W2
                gb = c * NN + base
                sv = rvN.at[pl.ds(0, W2), :]
                ev = rvN.at[pl.ds(W2, W2), :]
                bv1 = rvN.at[pl.ds(2 * W2, W2), :]
                bv2 = rvN.at[pl.ds(3 * W2, W2), :]
                pltpu.sync_copy(acc_sh.at[pl.ds(base, W2), :], sv)
                pltpu.sync_copy(
                    embp.at[pl.ds(base, W2), pl.ds(c * 32, 32)], ev)
                pltpu.sync_copy(g1.at[pl.ds(gb, W2), :], bv1)
                pltpu.sync_copy(g2.at[pl.ds(gb, W2), :], bv2)
                pltpu.sync_copy(deg_sh.at[pl.ds(base, W2)],
                                dvv.at[pl.ds(0, W2)])

                def out_step(gg, _):
                    d16 = dvv[pl.ds(gg * 16, 16)]
                    q16 = 0.25 / d16
                    d16q = d16 * 0.25
                    for i in range(16):
                        n = gg * 16 + i
                        d = jnp.full((16,), d16q[i])
                        q = jnp.full((16,), q16[i])
                        for h in (0, 16):
                            ev[n, pl.ds(h, 16)] = (
                                ev[n, pl.ds(h, 16)] * 0.25
                                + (bv1[n, pl.ds(h, 16)]
                                   + bv2[n, pl.ds(h, 16)]) * q
                                + sv[n, pl.ds(h, 16)] * d)
                    return 0

                lax.fori_loop(0, W2 // 16, out_step, 0)
                pltpu.sync_copy(ev, out.at[pl.ds(base, W2),
                                           pl.ds(c * 32, 32)])
                return 0

            lax.fori_loop(0, NK2, final_step, 0)


_mesh = plsc.VectorSubcoreMesh(core_axis_name="c", subcore_axis_name="s")

_sc_call = pl.kernel(
    _body,
    out_type=(
        jax.ShapeDtypeStruct((NN, 64), jnp.float32),
        jax.ShapeDtypeStruct((2 * NN, 32), jnp.float32),
        jax.ShapeDtypeStruct((2 * NN, 32), jnp.float32),
        jax.ShapeDtypeStruct((2 * NN, 32), jnp.float32),
    ),
    mesh=_mesh,
    compiler_params=pltpu.CompilerParams(use_tc_tiling_on_sc=False),
    scratch_types=[
        pltpu.VMEM_SHARED((NN, 32), jnp.float32),
        pltpu.VMEM_SHARED((NN,), jnp.float32),
        pltpu.VMEM((F * ECH, 32), jnp.float32),
        pltpu.VMEM((W,), jnp.float32),
        pltpu.VMEM((2 * F, ECH), jnp.int32),
        pltpu.VMEM((2 * F, ECH), jnp.int32),
        pltpu.VMEM((1, ECH), jnp.float32),
        pltpu.SemaphoreType.DMA,
        pltpu.SemaphoreType.DMA,
        pltpu.SemaphoreType.DMA,
        pltpu.SemaphoreType.DMA,
        pltpu.SemaphoreType.DMA,
    ],
)


@jax.jit
def kernel(edge_index, embedding_weight):
    padv = TRASH + (jnp.arange(EPAD - E, dtype=jnp.int32) % 176)
    eip = jnp.concatenate(
        [edge_index, jnp.stack([padv, padv])], axis=1)
    embp = jnp.pad(embedding_weight, ((0, NN - NV), (0, 0)))
    z2 = jnp.zeros((W, 32), jnp.float32)
    z1 = jnp.zeros((PT,), jnp.float32)
    out, _, _, _ = _sc_call(eip, embp, z2, z1)
    return out[:NV // 2], out[NV // 2:NV]


# speedup vs baseline: 1.8395x; 1.0315x over previous
import jax
import jax.numpy as jnp
from jax import lax
from jax.experimental import pallas as pl
from jax.experimental.pallas import tpu as pltpu
from jax.experimental.pallas import tpu_sc as plsc

NV = 50000
NN = 50176
PT = 3136
W = 448
NK = PT // W
W2 = 112
NK2 = PT // W2
E = 800000
ECH = 128
F = 4
NB = 98
CT = NB * F
EPAD = 16 * (CT + F) * ECH
TRASH = NV


def _body(ei, embp, z2, z1, out, g0, g1, g2,
          acc_sh, deg_sh, rvN, dvv, rowv, colv, onesv,
          isem, gsem, ssem, zsem, lsem):
    c = lax.axis_index("c")
    s = lax.axis_index("s")
    coffv = jnp.full((16,), c * NN, jnp.int32)

    def idx_src(which, ct):
        return ei.at[which, pl.ds((s + 16 * ct) * ECH, ECH)]

    def slotref(f):
        return rvN.at[pl.ds(f * ECH, ECH), :]

    def prefetch(g, h):
        ds = []
        for f in range(F):
            ct = g * F + f
            ds.append(pltpu.async_copy(idx_src(0, ct), rowv.at[h + f], isem))
            ds.append(pltpu.async_copy(idx_src(1, ct), colv.at[h + f], isem))
        return ds

    for t in range(8):
        onesv[0, pl.ds(16 * t, 16)] = jnp.full((16,), 1.0, jnp.float32)
    pltpu.sync_copy(z1, deg_sh.at[pl.ds(s * PT, PT)])
    plsc.subcore_barrier()

    for f in range(F):
        pltpu.sync_copy(idx_src(0, f), rowv.at[f])
        pltpu.sync_copy(idx_src(1, f), colv.at[f])

    def deg_step(g, _):
        h = lax.rem(g, 2) * F
        hn = F - h
        di = prefetch(g + 1, hn)
        sd = []
        for f in range(F):
            sd.append(pltpu.async_copy(
                onesv.at[0], deg_sh.at[rowv.at[h + f]], ssem, add=True))
            sd.append(pltpu.async_copy(
                onesv.at[0], deg_sh.at[colv.at[h + f]], ssem, add=True))
        for d in sd:
            d.wait()
        for d in di:
            d.wait()
        return 0

    lax.fori_loop(0, NB, deg_step, 0)
    plsc.subcore_barrier()

    def init_step(k, _):
        base = s * PT + k * W
        gb = c * NN + base
        ev = rvN.at[pl.ds(0, W), :]
        zd = pltpu.async_copy(z2, acc_sh.at[pl.ds(base, W), :], zsem)
        pltpu.sync_copy(deg_sh.at[pl.ds(base, W)], dvv)

        def rsqrt_step(gg, _):
            x = jnp.maximum(dvv[pl.ds(gg * 16, 16)], 1.0)
            bits = lax.bitcast_convert_type(x, jnp.int32)
            y = lax.bitcast_convert_type(
                jnp.int32(0x5F3759DF) - lax.shift_right_arithmetic(bits, 1),
                jnp.float32)
            half = x * 0.5
            for _ in range(3):
                y = y * (1.5 - half * y * y)
            dvv[pl.ds(gg * 16, 16)] = y
            return 0

        lax.fori_loop(0, W // 16, rsqrt_step, 0)
        pltpu.sync_copy(dvv, deg_sh.at[pl.ds(base, W)])
        pltpu.sync_copy(embp.at[pl.ds(base, W), pl.ds(c * 32, 32)], ev)

        def scale_step(gg, _):
            d16 = dvv[pl.ds(gg * 16, 16)]
            for i in range(16):
                n = gg * 16 + i
                d = jnp.full((16,), d16[i])
                ev[n, pl.ds(0, 16)] = ev[n, pl.ds(0, 16)] * d
                ev[n, pl.ds(16, 16)] = ev[n, pl.ds(16, 16)] * d
            return 0

        lax.fori_loop(0, W // 16, scale_step, 0)
        pltpu.sync_copy(ev, g0.at[pl.ds(gb, W), :])
        zd.wait()
        return 0

    lax.fori_loop(0, NK, init_step, 0)
    plsc.subcore_barrier()

    for layer, (gin, gout) in enumerate([(g0, g1), (g1, g2), (g2, None)]):
        last = gout is None

        for f in range(F):
            pltpu.sync_copy(idx_src(0, f), rowv.at[f])
            pltpu.sync_copy(idx_src(1, f), colv.at[f])

        def edge_step(g, _):
            h = lax.rem(g, 2) * F
            hn = F - h
            di = prefetch(g + 1, hn)
            gd = []
            for f in range(F):
                for t in range(8):
                    colv[h + f, pl.ds(16 * t, 16)] = (
                        colv[h + f, pl.ds(16 * t, 16)] + coffv)
                gd.append(pltpu.async_copy(
                    gin.at[colv.at[h + f]], slotref(f), gsem))
            sd = []
            for f in range(F):
                gd[f].wait()
                sd.append(pltpu.async_copy(
                    slotref(f), acc_sh.at[rowv.at[h + f]], ssem, add=True))
            for d in sd:
                d.wait()
            for d in di:
                d.wait()
            return 0

        lax.fori_loop(0, NB, edge_step, 0)
        plsc.subcore_barrier()

        if not last:
            def write_step(k, _):
                base = s * PT + k * W
                gb = c * NN + base
                sv = rvN.at[pl.ds(0, W), :]
                pltpu.sync_copy(acc_sh.at[pl.ds(base, W), :], sv)
                zd = pltpu.async_copy(z2, acc_sh.at[pl.ds(base, W), :], zsem)
                pltpu.sync_copy(deg_sh.at[pl.ds(base, W)], dvv)

                def out_step(gg, _):
                    d16 = dvv[pl.ds(gg * 16, 16)]
                    d16 = d16 * d16
                    for i in range(16):
                        n = gg * 16 + i
                        d = jnp.full((16,), d16[i])
                        sv[n, pl.ds(0, 16)] = sv[n, pl.ds(0, 16)] * d
                        sv[n, pl.ds(16, 16)] = sv[n, pl.ds(16, 16)] * d
                    return 0

                lax.fori_loop(0, W // 16, out_step, 0)
                pltpu.sync_copy(sv, gout.at[pl.ds(gb, W), :])
                zd.wait()
                return 0

            lax.fori_loop(0, NK, write_step, 0)
            plsc.subcore_barrier()
        else:
            def final_step(k, _):
                base = s * PT + k * W2
                gb = c * NN + base
                sv = rvN.at[pl.ds(0, W2), :]
                ev = rvN.at[pl.ds(W2, W2), :]
                bv1 = rvN.at[pl.ds(2 * W2, W2), :]
                bv2 = rvN.at[pl.ds(3 * W2, W2), :]
                pltpu.sync_copy(acc_sh.at[pl.ds(base, W2), :], sv)
                pltpu.sync_copy(
                    embp.at[pl.ds(base, W2), pl.ds(c * 32, 32)], ev)
                pltpu.sync_copy(g1.at[pl.ds(gb, W2), :], bv1)
                pltpu.sync_copy(g2.at[pl.ds(gb, W2), :], bv2)
                pltpu.sync_copy(deg_sh.at[pl.ds(base, W2)],
                                dvv.at[pl.ds(0, W2)])

                def out_step(gg, _):
                    d16 = dvv[pl.ds(gg * 16, 16)]
                    q16 = 0.25 / d16
                    d16q = d16 * 0.25
                    for i in range(16):
                        n = gg * 16 + i
                        d = jnp.full((16,), d16q[i])
                        q = jnp.full((16,), q16[i])
                        for h in (0, 16):
                            ev[n, pl.ds(h, 16)] = (
                                ev[n, pl.ds(h, 16)] * 0.25
                                + (bv1[n, pl.ds(h, 16)]
                                   + bv2[n, pl.ds(h, 16)]) * q
                                + sv[n, pl.ds(h, 16)] * d)
                    return 0

                lax.fori_loop(0, W2 // 16, out_step, 0)
                pltpu.sync_copy(ev, out.at[pl.ds(base, W2),
                                           pl.ds(c * 32, 32)])
                return 0

            lax.fori_loop(0, NK2, final_step, 0)


_mesh = plsc.VectorSubcoreMesh(core_axis_name="c", subcore_axis_name="s")

_sc_call = pl.kernel(
    _body,
    out_type=(
        jax.ShapeDtypeStruct((NN, 64), jnp.float32),
        jax.ShapeDtypeStruct((2 * NN, 32), jnp.float32),
        jax.ShapeDtypeStruct((2 * NN, 32), jnp.float32),
        jax.ShapeDtypeStruct((2 * NN, 32), jnp.float32),
    ),
    mesh=_mesh,
    compiler_params=pltpu.CompilerParams(use_tc_tiling_on_sc=False),
    scratch_types=[
        pltpu.VMEM_SHARED((NN, 32), jnp.float32),
        pltpu.VMEM_SHARED((NN,), jnp.float32),
        pltpu.VMEM((F * ECH, 32), jnp.float32),
        pltpu.VMEM((W,), jnp.float32),
        pltpu.VMEM((2 * F, ECH), jnp.int32),
        pltpu.VMEM((2 * F, ECH), jnp.int32),
        pltpu.VMEM((1, ECH), jnp.float32),
        pltpu.SemaphoreType.DMA,
        pltpu.SemaphoreType.DMA,
        pltpu.SemaphoreType.DMA,
        pltpu.SemaphoreType.DMA,
        pltpu.SemaphoreType.DMA,
    ],
)


@jax.jit
def kernel(edge_index, embedding_weight):
    padv = TRASH + (jnp.arange(EPAD - E, dtype=jnp.int32) % 176)
    eip = jnp.concatenate(
        [edge_index, jnp.stack([padv, padv])], axis=1)
    embp = jnp.pad(embedding_weight, ((0, NN - NV), (0, 0)))
    z2 = jnp.zeros((W, 32), jnp.float32)
    z1 = jnp.zeros((PT,), jnp.float32)
    out, _, _, _ = _sc_call(eip, embp, z2, z1)
    return out[:NV // 2], out[NV // 2:NV]
